# baseline (speedup 1.0000x reference)
import functools

import jax
import jax.numpy as jnp
from jax import lax
from jax.experimental import pallas as pl
from jax.experimental.pallas import tpu as pltpu
from jax.experimental.pallas import tpu_sc as plsc

B = 64
GH = GW = 26
A = 5
NB = 16
NCLS = 80
CH = 5 + NCLS
CELLS = GH * GW * A
PADC = 3456
GS = 16.0
SR, SL = 20, 169
THR = 0.4
NC, NS = 2, 16
NW = NC * NS
BPW = B // NW
LN2 = 0.6931471805599453


def _log16(x):
    bits = lax.bitcast_convert_type(x, jnp.int32)
    e = ((bits >> 23) & 0xFF) - 127
    m = lax.bitcast_convert_type((bits & 0x7FFFFF) | (127 << 23), jnp.float32)
    big = m > 1.4142135381698608
    m = jnp.where(big, m * 0.5, m)
    e = (e + jnp.where(big, 1, 0)).astype(jnp.float32)
    z = (m - 1.0) / (m + 1.0)
    z2 = z * z
    p = 1.0 + z2 * (0.333333343 + z2 * (0.2 + z2 * (0.142857149 + z2 * 0.111111112)))
    return e * LN2 + 2.0 * z * p


def _sc_body(y5, boxes_t, labels2, anc_b, out,
             boxes_v, labels_v, anc_v, rows_v, out_v, sem):
    wid = lax.axis_index("s") * NC + lax.axis_index("c")
    iota = lax.iota(jnp.int32, 16)
    pltpu.sync_copy(anc_b, anc_v)
    awv = anc_v[pl.ds(0, 16)]
    ahv = anc_v[pl.ds(16, 16)]

    def one_batch(lb, accs):
        acc0, acc1, acc2, acc3 = accs
        b = wid * BPW + lb
        pltpu.sync_copy(boxes_t.at[b], boxes_v)
        pltpu.sync_copy(labels2.at[b], labels_v)
        bx1 = boxes_v[pl.ds(0, 16)]
        by1 = boxes_v[pl.ds(16, 16)]
        bx2 = boxes_v[pl.ds(32, 16)]
        by2 = boxes_v[pl.ds(48, 16)]
        lbv = labels_v[:]
        bw = bx2 - bx1
        bh = by2 - by1
        areag = bw * bh
        cxg = (bx1 + bx2) * (0.5 / GS)
        cyg = (by1 + by2) * (0.5 / GS)
        cellx = cxg.astype(jnp.int32)
        celly = cyg.astype(jnp.int32)
        offx = cxg - cellx.astype(jnp.float32)
        offy = cyg - celly.astype(jnp.float32)
        best = jnp.full((16,), -1.0, jnp.float32)
        aw_sel = jnp.zeros((16,), jnp.float32)
        ah_sel = jnp.zeros((16,), jnp.float32)
        aidx = jnp.zeros((16,), jnp.int32)
        for a in range(A):
            awa = awv[a]
            aha = ahv[a]
            inter = jnp.minimum(awa, bw) * jnp.minimum(aha, bh)
            ioua = inter / (awa * aha + areag - inter + 1e-12)
            take = ioua > best
            best = jnp.where(take, ioua, best)
            aidx = jnp.where(take, a, aidx)
            aw_sel = jnp.where(take, awa, aw_sel)
            ah_sel = jnp.where(take, aha, ah_sel)
        key = (celly * GW + cellx) * A + aidx
        copies = []
        for i in range(NB):
            copies.append(pltpu.async_copy(
                y5.at[b, celly[i], cellx[i]], rows_v.at[i, pl.ds(0, A)], sem))
        tw = _log16(bw / aw_sel)
        th = _log16(bh / ah_sel)
        dup_cell = iota < 0
        dup_pair = iota < 0
        for j in range(1, NB):
            samek = (key == key[j]) & (iota < j)
            dup_cell = dup_cell | samek
            dup_pair = dup_pair | (samek & (lbv == lbv[j]))
        ucf = jnp.where(dup_cell, 0.0, 1.0)
        upf = jnp.where(dup_pair, 0.0, 1.0)
        for c in copies:
            c.wait()
        y0v = jnp.zeros((16,), jnp.float32)
        y1v = jnp.zeros((16,), jnp.float32)
        y2v = jnp.zeros((16,), jnp.float32)
        y3v = jnp.zeros((16,), jnp.float32)
        y4v = jnp.zeros((16,), jnp.float32)
        acc0 = acc0 + upf
        for i in range(NB):
            lane = iota == i
            pos = lbv[i] + 5
            ucf_i = ucf[i]
            upf_i = upf[i]
            ai = aidx[i]
            ssq = jnp.zeros((16,), jnp.float32)
            ylsel = jnp.zeros((16,), jnp.float32)
            for k in range(6):
                off = k * 16 if k < 5 else 69
                ch = rows_v[i, ai, pl.ds(off, 16)]
                if k == 0:
                    y0v = jnp.where(lane, ch[0], y0v)
                    y1v = jnp.where(lane, ch[1], y1v)
                    y2v = jnp.where(lane, ch[2], y2v)
                    y3v = jnp.where(lane, ch[3], y3v)
                    y4v = jnp.where(lane, ch[4], y4v)
                    chm = jnp.where(iota >= 5, ch, 0.0)
                    hit = iota == pos
                elif k == 5:
                    chm = jnp.where(iota >= 11, ch, 0.0)
                    hit = (iota >= 11) & (iota + 69 == pos)
                else:
                    chm = ch
                    hit = iota + off == pos
                ssq = ssq + chm * chm
                ylsel = ylsel + jnp.where(hit, ch, 0.0)
            acc0 = acc0 + ucf_i * ssq - (2.0 * upf_i) * ylsel
        pcx = (1.0 / (1.0 + jnp.exp(-y0v)) + cellx.astype(jnp.float32)) * GS
        pcy = (1.0 / (1.0 + jnp.exp(-y1v)) + celly.astype(jnp.float32)) * GS
        pw = aw_sel * jnp.exp(y2v)
        ph = ah_sel * jnp.exp(y3v)
        px1 = pcx - pw * 0.5
        px2 = pcx + pw * 0.5
        py1 = pcy - ph * 0.5
        py2 = pcy + ph * 0.5
        areap = pw * ph
        ioumax = jnp.full((16,), -1.0, jnp.float32)
        tiou = jnp.zeros((16,), jnp.float32)
        for j in range(NB):
            areagj = areag[j]
            iw = jnp.maximum(jnp.minimum(px2, bx2[j]) - jnp.maximum(px1, bx1[j]), 0.0)
            ih = jnp.maximum(jnp.minimum(py2, by2[j]) - jnp.maximum(py1, by1[j]), 0.0)
            inter = iw * ih
            iouj = inter / (areap + areagj - inter + 1e-12)
            ioumax = jnp.maximum(ioumax, iouj)
            tiou = jnp.where(iota == j, iouj, tiou)
        d0 = y0v - offx
        d1 = y1v - offy
        d2 = y2v - tw
        d3 = y3v - th
        d4 = y4v - tiou
        obj_terms = d0 * d0 + d1 * d1 + d2 * d2 + d3 * d3 + d4 * d4
        acc0 = acc0 + ucf * obj_terms
        acc1 = acc1 + ucf
        mcf = jnp.where(ioumax < THR, ucf, 0.0)
        acc2 = acc2 + mcf * y4v * y4v
        acc3 = acc3 + mcf
        return (acc0, acc1, acc2, acc3)

    z = jnp.zeros((16,), jnp.float32)
    acc0, acc1, acc2, acc3 = lax.fori_loop(0, BPW, one_batch, (z, z, z, z))
    out_v[0, :] = acc0
    out_v[1, :] = acc1
    out_v[2, :] = acc2
    out_v[3, :] = acc3
    pltpu.sync_copy(out_v, out.at[wid])


def _make_sc_obj():
    return functools.partial(
        pl.kernel,
        out_type=jax.ShapeDtypeStruct((NW, 4, 16), jnp.float32),
        mesh=plsc.VectorSubcoreMesh(core_axis_name="c", subcore_axis_name="s",
                                    num_cores=NC, num_subcores=NS),
        scratch_types=[
            pltpu.VMEM((64,), jnp.float32),
            pltpu.VMEM((16,), jnp.int32),
            pltpu.VMEM((32,), jnp.float32),
            pltpu.VMEM((16, 8, CH), jnp.float32),
            pltpu.VMEM((4, 16), jnp.float32),
            pltpu.SemaphoreType.DMA,
        ],
    )(_sc_body)


def _tc_body(t5_ref, boxes_ref, anc_ref, gx_ref, gy_ref, av_ref,
             scp_ref, out_ref, acc_ref):
    b = pl.program_id(0)

    @pl.when(b == 0)
    def _init():
        acc_ref[0] = 0.0
        acc_ref[1] = 0.0

    def rows2d(c):
        return jnp.stack([t5_ref[0, c, pl.ds(r * SL, SL)] for r in range(SR)])
    x0 = rows2d(0)
    x1 = rows2d(1)
    x2 = rows2d(2)
    x3 = rows2d(3)
    pi = rows2d(4)
    av = av_ref[...]
    aw = jnp.zeros((SR, SL), jnp.float32)
    ah = jnp.zeros((SR, SL), jnp.float32)
    for a in range(A):
        s = av == float(a)
        aw = jnp.where(s, anc_ref[a, 0], aw)
        ah = jnp.where(s, anc_ref[a, 1], ah)
    pcx = (jax.nn.sigmoid(x0) + gx_ref[...]) * GS
    pcy = (jax.nn.sigmoid(x1) + gy_ref[...]) * GS
    pw = aw * jnp.exp(x2)
    ph = ah * jnp.exp(x3)
    px1 = pcx - pw * 0.5
    px2 = pcx + pw * 0.5
    py1 = pcy - ph * 0.5
    py2 = pcy + ph * 0.5
    areap = pw * ph
    noobj = av > -1.0
    rhs0 = THR * areap
    for j in range(NB):
        bx1j = boxes_ref[0, 0, j]
        by1j = boxes_ref[0, 1, j]
        bx2j = boxes_ref[0, 2, j]
        by2j = boxes_ref[0, 3, j]
        areagj = (bx2j - bx1j) * (by2j - by1j)
        iw = jnp.maximum(jnp.minimum(px2, bx2j) - jnp.maximum(px1, bx1j), 0.0)
        ih = jnp.maximum(jnp.minimum(py2, by2j) - jnp.maximum(py1, by1j), 0.0)
        inter = iw * ih
        noobj = noobj & ((1.0 + THR) * inter < rhs0 + THR * areagj)
    acc_ref[0] += jnp.sum(jnp.where(noobj, pi * pi, 0.0))
    acc_ref[1] += jnp.sum(jnp.where(noobj, 1.0, 0.0))

    @pl.when(b == B - 1)
    def _fin():
        p = scp_ref[...]
        sobj = jnp.sum(p[:, 0, :])
        n1 = jnp.sum(p[:, 1, :])
        cs = jnp.sum(p[:, 2, :])
        cn = jnp.sum(p[:, 3, :])
        out_ref[0, 0] = sobj / n1 + (acc_ref[0] - cs) / (acc_ref[1] - cn)


def kernel(yolo_output, boxes, labels, anchor):
    t5 = jnp.moveaxis(yolo_output, -1, 1).reshape(B, CH, CELLS)
    boxes_t = jnp.swapaxes(boxes, 1, 2)
    anc_b = jnp.pad(anchor.T, ((0, 0), (0, 16 - A))).reshape(32)
    k = jnp.arange(CELLS, dtype=jnp.int32)
    gxc = ((k // A) % GW).astype(jnp.float32).reshape(SR, SL)
    gyc = (k // (GW * A)).astype(jnp.float32).reshape(SR, SL)
    avc = (k % A).astype(jnp.float32).reshape(SR, SL)

    scp = _make_sc_obj()(yolo_output, boxes_t.reshape(B, 64), labels, anc_b)

    loss = pl.pallas_call(
        _tc_body,
        grid=(B,),
        in_specs=[
            pl.BlockSpec((1, 8, CELLS), lambda b: (b, 0, 0)),
            pl.BlockSpec((1, 4, NB), lambda b: (b, 0, 0),
                         memory_space=pltpu.SMEM),
            pl.BlockSpec((A, 2), lambda b: (0, 0), memory_space=pltpu.SMEM),
            pl.BlockSpec((SR, SL), lambda b: (0, 0)),
            pl.BlockSpec((SR, SL), lambda b: (0, 0)),
            pl.BlockSpec((SR, SL), lambda b: (0, 0)),
            pl.BlockSpec((NW, 4, 16), lambda b: (0, 0, 0)),
        ],
        out_specs=pl.BlockSpec((1, 1), lambda b: (0, 0),
                               memory_space=pltpu.SMEM),
        out_shape=jax.ShapeDtypeStruct((1, 1), jnp.float32),
        scratch_shapes=[pltpu.SMEM((2,), jnp.float32)],
    )(t5, boxes_t, anchor, gxc, gyc, avc, scp)
    return loss[0, 0]


# speedup vs baseline: 1.1219x; 1.1219x over previous
import functools

import jax
import jax.numpy as jnp
from jax import lax
from jax.experimental import pallas as pl
from jax.experimental.pallas import tpu as pltpu
from jax.experimental.pallas import tpu_sc as plsc

B = 64
GH = GW = 26
A = 5
NB = 16
NCLS = 80
CH = 5 + NCLS
CELLS = GH * GW * A
PADC = 3456
GS = 16.0
SR, SL = 20, 169
THR = 0.4
NC, NS = 2, 16
NW = NC * NS
BPW = B // NW
LN2 = 0.6931471805599453


def _log16(x):
    bits = lax.bitcast_convert_type(x, jnp.int32)
    e = ((bits >> 23) & 0xFF) - 127
    m = lax.bitcast_convert_type((bits & 0x7FFFFF) | (127 << 23), jnp.float32)
    big = m > 1.4142135381698608
    m = jnp.where(big, m * 0.5, m)
    e = (e + jnp.where(big, 1, 0)).astype(jnp.float32)
    z = (m - 1.0) / (m + 1.0)
    z2 = z * z
    p = 1.0 + z2 * (0.333333343 + z2 * (0.2 + z2 * (0.142857149 + z2 * 0.111111112)))
    return e * LN2 + 2.0 * z * p


def _sc_body(y5, boxes_t, labels2, anc_b, out,
             boxes_v, labels_v, anc_v, rows_v, out_v, sem):
    wid = lax.axis_index("s") * NC + lax.axis_index("c")
    iota = lax.iota(jnp.int32, 16)
    pltpu.sync_copy(anc_b, anc_v)
    awv = anc_v[pl.ds(0, 16)]
    ahv = anc_v[pl.ds(16, 16)]

    def one_batch(lb, accs):
        acc0, acc1, acc2, acc3 = accs
        b = wid * BPW + lb
        pltpu.sync_copy(boxes_t.at[b], boxes_v)
        pltpu.sync_copy(labels2.at[b], labels_v)
        bx1 = boxes_v[pl.ds(0, 16)]
        by1 = boxes_v[pl.ds(16, 16)]
        bx2 = boxes_v[pl.ds(32, 16)]
        by2 = boxes_v[pl.ds(48, 16)]
        lbv = labels_v[:]
        bw = bx2 - bx1
        bh = by2 - by1
        areag = bw * bh
        cxg = (bx1 + bx2) * (0.5 / GS)
        cyg = (by1 + by2) * (0.5 / GS)
        cellx = cxg.astype(jnp.int32)
        celly = cyg.astype(jnp.int32)
        offx = cxg - cellx.astype(jnp.float32)
        offy = cyg - celly.astype(jnp.float32)
        best = jnp.full((16,), -1.0, jnp.float32)
        aw_sel = jnp.zeros((16,), jnp.float32)
        ah_sel = jnp.zeros((16,), jnp.float32)
        aidx = jnp.zeros((16,), jnp.int32)
        for a in range(A):
            awa = awv[a]
            aha = ahv[a]
            inter = jnp.minimum(awa, bw) * jnp.minimum(aha, bh)
            ioua = inter / (awa * aha + areag - inter + 1e-12)
            take = ioua > best
            best = jnp.where(take, ioua, best)
            aidx = jnp.where(take, a, aidx)
            aw_sel = jnp.where(take, awa, aw_sel)
            ah_sel = jnp.where(take, aha, ah_sel)
        key = (celly * GW + cellx) * A + aidx
        copies = []
        for i in range(NB):
            copies.append(pltpu.async_copy(
                y5.at[b, celly[i], cellx[i]], rows_v.at[i, pl.ds(0, A)], sem))
        tw = _log16(bw / aw_sel)
        th = _log16(bh / ah_sel)
        dup_cell = iota < 0
        dup_pair = iota < 0
        for j in range(1, NB):
            samek = (key == key[j]) & (iota < j)
            dup_cell = dup_cell | samek
            dup_pair = dup_pair | (samek & (lbv == lbv[j]))
        ucf = jnp.where(dup_cell, 0.0, 1.0)
        upf = jnp.where(dup_pair, 0.0, 1.0)
        for c in copies:
            c.wait()
        y0v = jnp.zeros((16,), jnp.float32)
        y1v = jnp.zeros((16,), jnp.float32)
        y2v = jnp.zeros((16,), jnp.float32)
        y3v = jnp.zeros((16,), jnp.float32)
        y4v = jnp.zeros((16,), jnp.float32)
        acc0 = acc0 + upf
        for i in range(NB):
            lane = iota == i
            pos = lbv[i] + 5
            ucf_i = ucf[i]
            upf_i = upf[i]
            ai = aidx[i]
            ssq = jnp.zeros((16,), jnp.float32)
            ylsel = jnp.zeros((16,), jnp.float32)
            for k in range(6):
                off = k * 16 if k < 5 else 69
                ch = rows_v[i, ai, pl.ds(off, 16)]
                if k == 0:
                    y0v = jnp.where(lane, ch[0], y0v)
                    y1v = jnp.where(lane, ch[1], y1v)
                    y2v = jnp.where(lane, ch[2], y2v)
                    y3v = jnp.where(lane, ch[3], y3v)
                    y4v = jnp.where(lane, ch[4], y4v)
                    chm = jnp.where(iota >= 5, ch, 0.0)
                    hit = iota == pos
                elif k == 5:
                    chm = jnp.where(iota >= 11, ch, 0.0)
                    hit = (iota >= 11) & (iota + 69 == pos)
                else:
                    chm = ch
                    hit = iota + off == pos
                ssq = ssq + chm * chm
                ylsel = ylsel + jnp.where(hit, ch, 0.0)
            acc0 = acc0 + ucf_i * ssq - (2.0 * upf_i) * ylsel
        pcx = (1.0 / (1.0 + jnp.exp(-y0v)) + cellx.astype(jnp.float32)) * GS
        pcy = (1.0 / (1.0 + jnp.exp(-y1v)) + celly.astype(jnp.float32)) * GS
        pw = aw_sel * jnp.exp(y2v)
        ph = ah_sel * jnp.exp(y3v)
        px1 = pcx - pw * 0.5
        px2 = pcx + pw * 0.5
        py1 = pcy - ph * 0.5
        py2 = pcy + ph * 0.5
        areap = pw * ph
        ioumax = jnp.full((16,), -1.0, jnp.float32)
        tiou = jnp.zeros((16,), jnp.float32)
        for j in range(NB):
            areagj = areag[j]
            iw = jnp.maximum(jnp.minimum(px2, bx2[j]) - jnp.maximum(px1, bx1[j]), 0.0)
            ih = jnp.maximum(jnp.minimum(py2, by2[j]) - jnp.maximum(py1, by1[j]), 0.0)
            inter = iw * ih
            iouj = inter / (areap + areagj - inter + 1e-12)
            ioumax = jnp.maximum(ioumax, iouj)
            tiou = jnp.where(iota == j, iouj, tiou)
        d0 = y0v - offx
        d1 = y1v - offy
        d2 = y2v - tw
        d3 = y3v - th
        d4 = y4v - tiou
        obj_terms = d0 * d0 + d1 * d1 + d2 * d2 + d3 * d3 + d4 * d4
        acc0 = acc0 + ucf * obj_terms
        acc1 = acc1 + ucf
        mcf = jnp.where(ioumax < THR, ucf, 0.0)
        acc2 = acc2 + mcf * y4v * y4v
        acc3 = acc3 + mcf
        return (acc0, acc1, acc2, acc3)

    z = jnp.zeros((16,), jnp.float32)
    acc0, acc1, acc2, acc3 = lax.fori_loop(0, BPW, one_batch, (z, z, z, z))
    out_v[0, :] = acc0
    out_v[1, :] = acc1
    out_v[2, :] = acc2
    out_v[3, :] = acc3
    pltpu.sync_copy(out_v, out.at[wid])


def _make_sc_obj():
    return functools.partial(
        pl.kernel,
        out_type=jax.ShapeDtypeStruct((NW, 4, 16), jnp.float32),
        mesh=plsc.VectorSubcoreMesh(core_axis_name="c", subcore_axis_name="s",
                                    num_cores=NC, num_subcores=NS),
        scratch_types=[
            pltpu.VMEM((64,), jnp.float32),
            pltpu.VMEM((16,), jnp.int32),
            pltpu.VMEM((32,), jnp.float32),
            pltpu.VMEM((16, 8, CH), jnp.float32),
            pltpu.VMEM((4, 16), jnp.float32),
            pltpu.SemaphoreType.DMA,
        ],
    )(_sc_body)


def _tc_body(t5_ref, boxes_ref, anc_ref, gx_ref, gy_ref, av_ref,
             msk_ref, scp_ref, out_ref, acc_ref):
    b = pl.program_id(0)

    @pl.when(b == 0)
    def _init():
        acc_ref[0] = 0.0
        acc_ref[1] = 0.0

    x0 = t5_ref[0, 0]
    x1 = t5_ref[0, 1]
    x2 = t5_ref[0, 2]
    x3 = t5_ref[0, 3]
    pi = t5_ref[0, 4]
    av = av_ref[...]
    aw = jnp.zeros((27, 128), jnp.float32)
    ah = jnp.zeros((27, 128), jnp.float32)
    for a in range(A):
        s = av == float(a)
        aw = jnp.where(s, anc_ref[a, 0], aw)
        ah = jnp.where(s, anc_ref[a, 1], ah)
    pcx = (jax.nn.sigmoid(x0) + gx_ref[...]) * GS
    pcy = (jax.nn.sigmoid(x1) + gy_ref[...]) * GS
    pw = aw * jnp.exp(x2)
    ph = ah * jnp.exp(x3)
    px1 = pcx - pw * 0.5
    px2 = pcx + pw * 0.5
    py1 = pcy - ph * 0.5
    py2 = pcy + ph * 0.5
    areap = pw * ph
    noobj = msk_ref[...] > 0.5
    rhs0 = THR * areap
    for j in range(NB):
        bx1j = boxes_ref[0, 0, j]
        by1j = boxes_ref[0, 1, j]
        bx2j = boxes_ref[0, 2, j]
        by2j = boxes_ref[0, 3, j]
        areagj = (bx2j - bx1j) * (by2j - by1j)
        iw = jnp.maximum(jnp.minimum(px2, bx2j) - jnp.maximum(px1, bx1j), 0.0)
        ih = jnp.maximum(jnp.minimum(py2, by2j) - jnp.maximum(py1, by1j), 0.0)
        inter = iw * ih
        noobj = noobj & ((1.0 + THR) * inter < rhs0 + THR * areagj)
    acc_ref[0] += jnp.sum(jnp.where(noobj, pi * pi, 0.0))
    acc_ref[1] += jnp.sum(jnp.where(noobj, 1.0, 0.0))

    @pl.when(b == B - 1)
    def _fin():
        p = scp_ref[...]
        sobj = jnp.sum(p[:, 0, :])
        n1 = jnp.sum(p[:, 1, :])
        cs = jnp.sum(p[:, 2, :])
        cn = jnp.sum(p[:, 3, :])
        out_ref[0, 0] = sobj / n1 + (acc_ref[0] - cs) / (acc_ref[1] - cn)


def kernel(yolo_output, boxes, labels, anchor):
    t5 = jnp.zeros((B, 5, PADC), jnp.float32).at[:, :, :CELLS].set(
        jnp.moveaxis(yolo_output[..., :5], -1, 1).reshape(B, 5, CELLS)
    ).reshape(B, 5, 27, 128)
    boxes_t = jnp.swapaxes(boxes, 1, 2)
    anc_b = jnp.pad(anchor.T, ((0, 0), (0, 16 - A))).reshape(32)
    k = jnp.arange(PADC, dtype=jnp.int32)
    gxc = ((k // A) % GW).astype(jnp.float32).reshape(27, 128)
    gyc = (k // (GW * A)).astype(jnp.float32).reshape(27, 128)
    avc = (k % A).astype(jnp.float32).reshape(27, 128)
    mskc = (k < CELLS).astype(jnp.float32).reshape(27, 128)

    scp = _make_sc_obj()(yolo_output, boxes_t.reshape(B, 64), labels, anc_b)

    loss = pl.pallas_call(
        _tc_body,
        grid=(B,),
        in_specs=[
            pl.BlockSpec((1, 5, 27, 128), lambda b: (b, 0, 0, 0)),
            pl.BlockSpec((1, 4, NB), lambda b: (b, 0, 0),
                         memory_space=pltpu.SMEM),
            pl.BlockSpec((A, 2), lambda b: (0, 0), memory_space=pltpu.SMEM),
            pl.BlockSpec((27, 128), lambda b: (0, 0)),
            pl.BlockSpec((27, 128), lambda b: (0, 0)),
            pl.BlockSpec((27, 128), lambda b: (0, 0)),
            pl.BlockSpec((27, 128), lambda b: (0, 0)),
            pl.BlockSpec((NW, 4, 16), lambda b: (0, 0, 0)),
        ],
        out_specs=pl.BlockSpec((1, 1), lambda b: (0, 0),
                               memory_space=pltpu.SMEM),
        out_shape=jax.ShapeDtypeStruct((1, 1), jnp.float32),
        scratch_shapes=[pltpu.SMEM((2,), jnp.float32)],
    )(t5, boxes_t, anchor, gxc, gyc, avc, mskc, scp)
    return loss[0, 0]


# speedup vs baseline: 1.2050x; 1.0741x over previous
import functools

import jax
import jax.numpy as jnp
from jax import lax
from jax.experimental import pallas as pl
from jax.experimental.pallas import tpu as pltpu
from jax.experimental.pallas import tpu_sc as plsc

B = 64
GH = GW = 26
A = 5
NB = 16
NCLS = 80
CH = 5 + NCLS
CELLS = GH * GW * A
PADC = 3456
GS = 16.0
SR, SL = 20, 169
THR = 0.4
NC, NS = 2, 16
NW = NC * NS
BPW = B // NW
LN2 = 0.6931471805599453


def _log16(x):
    bits = lax.bitcast_convert_type(x, jnp.int32)
    e = ((bits >> 23) & 0xFF) - 127
    m = lax.bitcast_convert_type((bits & 0x7FFFFF) | (127 << 23), jnp.float32)
    big = m > 1.4142135381698608
    m = jnp.where(big, m * 0.5, m)
    e = (e + jnp.where(big, 1, 0)).astype(jnp.float32)
    z = (m - 1.0) / (m + 1.0)
    z2 = z * z
    p = 1.0 + z2 * (0.333333343 + z2 * (0.2 + z2 * (0.142857149 + z2 * 0.111111112)))
    return e * LN2 + 2.0 * z * p


def _sc_body(y5, boxes_t, labels2, anc_b, out,
             boxes_v, labels_v, anc_v, rows_v, out_v, sem):
    wid = lax.axis_index("s") * NC + lax.axis_index("c")
    iota = lax.iota(jnp.int32, 16)
    pltpu.sync_copy(anc_b, anc_v)
    awv = anc_v[pl.ds(0, 16)]
    ahv = anc_v[pl.ds(16, 16)]

    def one_batch(lb, accs):
        acc0, acc1, acc2, acc3 = accs
        b = wid * BPW + lb
        pltpu.sync_copy(boxes_t.at[b], boxes_v)
        pltpu.sync_copy(labels2.at[b], labels_v)
        bx1 = boxes_v[pl.ds(0, 16)]
        by1 = boxes_v[pl.ds(16, 16)]
        bx2 = boxes_v[pl.ds(32, 16)]
        by2 = boxes_v[pl.ds(48, 16)]
        lbv = labels_v[:]
        bw = bx2 - bx1
        bh = by2 - by1
        areag = bw * bh
        cxg = (bx1 + bx2) * (0.5 / GS)
        cyg = (by1 + by2) * (0.5 / GS)
        cellx = cxg.astype(jnp.int32)
        celly = cyg.astype(jnp.int32)
        offx = cxg - cellx.astype(jnp.float32)
        offy = cyg - celly.astype(jnp.float32)
        best = jnp.full((16,), -1.0, jnp.float32)
        aw_sel = jnp.zeros((16,), jnp.float32)
        ah_sel = jnp.zeros((16,), jnp.float32)
        aidx = jnp.zeros((16,), jnp.int32)
        for a in range(A):
            awa = awv[a]
            aha = ahv[a]
            inter = jnp.minimum(awa, bw) * jnp.minimum(aha, bh)
            ioua = inter / (awa * aha + areag - inter + 1e-12)
            take = ioua > best
            best = jnp.where(take, ioua, best)
            aidx = jnp.where(take, a, aidx)
            aw_sel = jnp.where(take, awa, aw_sel)
            ah_sel = jnp.where(take, aha, ah_sel)
        key = (celly * GW + cellx) * A + aidx
        copies = []
        for i in range(NB):
            copies.append(pltpu.async_copy(
                y5.at[b, celly[i], cellx[i]], rows_v.at[i, pl.ds(0, A)], sem))
        tw = _log16(bw / aw_sel)
        th = _log16(bh / ah_sel)
        dup_cell = iota < 0
        dup_pair = iota < 0
        for j in range(1, NB):
            samek = (key == key[j]) & (iota < j)
            dup_cell = dup_cell | samek
            dup_pair = dup_pair | (samek & (lbv == lbv[j]))
        ucf = jnp.where(dup_cell, 0.0, 1.0)
        upf = jnp.where(dup_pair, 0.0, 1.0)
        for c in copies:
            c.wait()
        y0v = jnp.zeros((16,), jnp.float32)
        y1v = jnp.zeros((16,), jnp.float32)
        y2v = jnp.zeros((16,), jnp.float32)
        y3v = jnp.zeros((16,), jnp.float32)
        y4v = jnp.zeros((16,), jnp.float32)
        acc0 = acc0 + upf
        for i in range(NB):
            lane = iota == i
            pos = lbv[i] + 5
            ucf_i = ucf[i]
            upf_i = upf[i]
            ai = aidx[i]
            ssq = jnp.zeros((16,), jnp.float32)
            ylsel = jnp.zeros((16,), jnp.float32)
            for k in range(6):
                off = k * 16 if k < 5 else 69
                ch = rows_v[i, ai, pl.ds(off, 16)]
                if k == 0:
                    y0v = jnp.where(lane, ch[0], y0v)
                    y1v = jnp.where(lane, ch[1], y1v)
                    y2v = jnp.where(lane, ch[2], y2v)
                    y3v = jnp.where(lane, ch[3], y3v)
                    y4v = jnp.where(lane, ch[4], y4v)
                    chm = jnp.where(iota >= 5, ch, 0.0)
                    hit = iota == pos
                elif k == 5:
                    chm = jnp.where(iota >= 11, ch, 0.0)
                    hit = (iota >= 11) & (iota + 69 == pos)
                else:
                    chm = ch
                    hit = iota + off == pos
                ssq = ssq + chm * chm
                ylsel = ylsel + jnp.where(hit, ch, 0.0)
            acc0 = acc0 + ucf_i * ssq - (2.0 * upf_i) * ylsel
        pcx = (1.0 / (1.0 + jnp.exp(-y0v)) + cellx.astype(jnp.float32)) * GS
        pcy = (1.0 / (1.0 + jnp.exp(-y1v)) + celly.astype(jnp.float32)) * GS
        pw = aw_sel * jnp.exp(y2v)
        ph = ah_sel * jnp.exp(y3v)
        px1 = pcx - pw * 0.5
        px2 = pcx + pw * 0.5
        py1 = pcy - ph * 0.5
        py2 = pcy + ph * 0.5
        areap = pw * ph
        ioumax = jnp.full((16,), -1.0, jnp.float32)
        tiou = jnp.zeros((16,), jnp.float32)
        for j in range(NB):
            areagj = areag[j]
            iw = jnp.maximum(jnp.minimum(px2, bx2[j]) - jnp.maximum(px1, bx1[j]), 0.0)
            ih = jnp.maximum(jnp.minimum(py2, by2[j]) - jnp.maximum(py1, by1[j]), 0.0)
            inter = iw * ih
            iouj = inter / (areap + areagj - inter + 1e-12)
            ioumax = jnp.maximum(ioumax, iouj)
            tiou = jnp.where(iota == j, iouj, tiou)
        d0 = y0v - offx
        d1 = y1v - offy
        d2 = y2v - tw
        d3 = y3v - th
        d4 = y4v - tiou
        obj_terms = d0 * d0 + d1 * d1 + d2 * d2 + d3 * d3 + d4 * d4
        acc0 = acc0 + ucf * obj_terms
        acc1 = acc1 + ucf
        mcf = jnp.where(ioumax < THR, ucf, 0.0)
        acc2 = acc2 + mcf * y4v * y4v
        acc3 = acc3 + mcf
        return (acc0, acc1, acc2, acc3)

    z = jnp.zeros((16,), jnp.float32)
    acc0, acc1, acc2, acc3 = lax.fori_loop(0, BPW, one_batch, (z, z, z, z))
    out_v[0, :] = acc0
    out_v[1, :] = acc1
    out_v[2, :] = acc2
    out_v[3, :] = acc3
    pltpu.sync_copy(out_v, out.at[wid])


def _make_sc_obj():
    return functools.partial(
        pl.kernel,
        out_type=jax.ShapeDtypeStruct((NW, 4, 16), jnp.float32),
        mesh=plsc.VectorSubcoreMesh(core_axis_name="c", subcore_axis_name="s",
                                    num_cores=NC, num_subcores=NS),
        scratch_types=[
            pltpu.VMEM((64,), jnp.float32),
            pltpu.VMEM((16,), jnp.int32),
            pltpu.VMEM((32,), jnp.float32),
            pltpu.VMEM((16, 8, CH), jnp.float32),
            pltpu.VMEM((4, 16), jnp.float32),
            pltpu.SemaphoreType.DMA,
        ],
    )(_sc_body)


def _tc_body(t5_ref, boxes_ref, anc_ref, gx_ref, gy_ref, av_ref,
             msk_ref, scp_ref, out_ref, acc_ref):
    b = pl.program_id(0)

    @pl.when(b == 0)
    def _init():
        acc_ref[0] = 0.0
        acc_ref[1] = 0.0

    av = av_ref[...]
    aw = jnp.zeros((27, 128), jnp.float32)
    ah = jnp.zeros((27, 128), jnp.float32)
    for a in range(A):
        s = av == float(a)
        aw = jnp.where(s, anc_ref[a, 0], aw)
        ah = jnp.where(s, anc_ref[a, 1], ah)
    for lb in range(2):
        x0 = t5_ref[lb, 0]
        x1 = t5_ref[lb, 1]
        x2 = t5_ref[lb, 2]
        x3 = t5_ref[lb, 3]
        pi = t5_ref[lb, 4]
        pcx = (jax.nn.sigmoid(x0) + gx_ref[...]) * GS
        pcy = (jax.nn.sigmoid(x1) + gy_ref[...]) * GS
        pw = aw * jnp.exp(x2)
        ph = ah * jnp.exp(x3)
        px1 = pcx - pw * 0.5
        px2 = pcx + pw * 0.5
        py1 = pcy - ph * 0.5
        py2 = pcy + ph * 0.5
        areap = pw * ph
        noobj = msk_ref[...] > 0.5
        rhs0 = THR * areap
        for j in range(NB):
            bx1j = boxes_ref[lb, 0, j]
            by1j = boxes_ref[lb, 1, j]
            bx2j = boxes_ref[lb, 2, j]
            by2j = boxes_ref[lb, 3, j]
            areagj = (bx2j - bx1j) * (by2j - by1j)
            iw = jnp.maximum(jnp.minimum(px2, bx2j) - jnp.maximum(px1, bx1j), 0.0)
            ih = jnp.maximum(jnp.minimum(py2, by2j) - jnp.maximum(py1, by1j), 0.0)
            inter = iw * ih
            noobj = noobj & ((1.0 + THR) * inter < rhs0 + THR * areagj)
        acc_ref[0] += jnp.sum(jnp.where(noobj, pi * pi, 0.0))
        acc_ref[1] += jnp.sum(jnp.where(noobj, 1.0, 0.0))

    @pl.when(b == B // 2 - 1)
    def _fin():
        p = scp_ref[...]
        sobj = jnp.sum(p[:, 0, :])
        n1 = jnp.sum(p[:, 1, :])
        cs = jnp.sum(p[:, 2, :])
        cn = jnp.sum(p[:, 3, :])
        out_ref[0, 0] = sobj / n1 + (acc_ref[0] - cs) / (acc_ref[1] - cn)


def kernel(yolo_output, boxes, labels, anchor):
    t5 = jnp.zeros((B, 5, PADC), jnp.float32).at[:, :, :CELLS].set(
        jnp.moveaxis(yolo_output[..., :5], -1, 1).reshape(B, 5, CELLS)
    ).reshape(B, 5, 27, 128)
    boxes_t = jnp.swapaxes(boxes, 1, 2)
    anc_b = jnp.pad(anchor.T, ((0, 0), (0, 16 - A))).reshape(32)
    k = jnp.arange(PADC, dtype=jnp.int32)
    gxc = ((k // A) % GW).astype(jnp.float32).reshape(27, 128)
    gyc = (k // (GW * A)).astype(jnp.float32).reshape(27, 128)
    avc = (k % A).astype(jnp.float32).reshape(27, 128)
    mskc = (k < CELLS).astype(jnp.float32).reshape(27, 128)

    scp = _make_sc_obj()(yolo_output, boxes_t.reshape(B, 64), labels, anc_b)

    loss = pl.pallas_call(
        _tc_body,
        grid=(B // 2,),
        in_specs=[
            pl.BlockSpec((2, 5, 27, 128), lambda b: (b, 0, 0, 0)),
            pl.BlockSpec((2, 4, NB), lambda b: (b, 0, 0),
                         memory_space=pltpu.SMEM),
            pl.BlockSpec((A, 2), lambda b: (0, 0), memory_space=pltpu.SMEM),
            pl.BlockSpec((27, 128), lambda b: (0, 0)),
            pl.BlockSpec((27, 128), lambda b: (0, 0)),
            pl.BlockSpec((27, 128), lambda b: (0, 0)),
            pl.BlockSpec((27, 128), lambda b: (0, 0)),
            pl.BlockSpec((NW, 4, 16), lambda b: (0, 0, 0)),
        ],
        out_specs=pl.BlockSpec((1, 1), lambda b: (0, 0),
                               memory_space=pltpu.SMEM),
        out_shape=jax.ShapeDtypeStruct((1, 1), jnp.float32),
        scratch_shapes=[pltpu.SMEM((2,), jnp.float32)],
    )(t5, boxes_t, anchor, gxc, gyc, avc, mskc, scp)
    return loss[0, 0]


# speedup vs baseline: 1.2540x; 1.0406x over previous
import functools

import jax
import jax.numpy as jnp
from jax import lax
from jax.experimental import pallas as pl
from jax.experimental.pallas import tpu as pltpu
from jax.experimental.pallas import tpu_sc as plsc

B = 64
GH = GW = 26
A = 5
NB = 16
NCLS = 80
CH = 5 + NCLS
CELLS = GH * GW * A
PADC = 3456
GS = 16.0
SR, SL = 20, 169
THR = 0.4
NC, NS = 2, 16
NW = NC * NS
BPW = B // NW
LN2 = 0.6931471805599453


def _log16(x):
    bits = lax.bitcast_convert_type(x, jnp.int32)
    e = ((bits >> 23) & 0xFF) - 127
    m = lax.bitcast_convert_type((bits & 0x7FFFFF) | (127 << 23), jnp.float32)
    big = m > 1.4142135381698608
    m = jnp.where(big, m * 0.5, m)
    e = (e + jnp.where(big, 1, 0)).astype(jnp.float32)
    z = (m - 1.0) / (m + 1.0)
    z2 = z * z
    p = 1.0 + z2 * (0.333333343 + z2 * (0.2 + z2 * (0.142857149 + z2 * 0.111111112)))
    return e * LN2 + 2.0 * z * p


def _sc_body(y5, boxes_t, labels2, anc_b, out,
             boxes_v, labels_v, anc_v, rows_v, out_v, sem):
    wid = lax.axis_index("s") * NC + lax.axis_index("c")
    iota = lax.iota(jnp.int32, 16)
    pltpu.sync_copy(anc_b, anc_v)
    awv = anc_v[pl.ds(0, 16)]
    ahv = anc_v[pl.ds(16, 16)]

    def one_batch(lb, accs):
        acc0, acc1, acc2, acc3 = accs
        b = wid * BPW + lb
        pltpu.sync_copy(boxes_t.at[b], boxes_v)
        pltpu.sync_copy(labels2.at[b], labels_v)
        bx1 = boxes_v[pl.ds(0, 16)]
        by1 = boxes_v[pl.ds(16, 16)]
        bx2 = boxes_v[pl.ds(32, 16)]
        by2 = boxes_v[pl.ds(48, 16)]
        lbv = labels_v[:]
        bw = bx2 - bx1
        bh = by2 - by1
        areag = bw * bh
        cxg = (bx1 + bx2) * (0.5 / GS)
        cyg = (by1 + by2) * (0.5 / GS)
        cellx = cxg.astype(jnp.int32)
        celly = cyg.astype(jnp.int32)
        offx = cxg - cellx.astype(jnp.float32)
        offy = cyg - celly.astype(jnp.float32)
        best = jnp.full((16,), -1.0, jnp.float32)
        aw_sel = jnp.zeros((16,), jnp.float32)
        ah_sel = jnp.zeros((16,), jnp.float32)
        aidx = jnp.zeros((16,), jnp.int32)
        for a in range(A):
            awa = awv[a]
            aha = ahv[a]
            inter = jnp.minimum(awa, bw) * jnp.minimum(aha, bh)
            ioua = inter / (awa * aha + areag - inter + 1e-12)
            take = ioua > best
            best = jnp.where(take, ioua, best)
            aidx = jnp.where(take, a, aidx)
            aw_sel = jnp.where(take, awa, aw_sel)
            ah_sel = jnp.where(take, aha, ah_sel)
        key = (celly * GW + cellx) * A + aidx
        copies = []
        for i in range(NB):
            copies.append(pltpu.async_copy(
                y5.at[b, celly[i], cellx[i]], rows_v.at[i, pl.ds(0, A)], sem))
        tw = _log16(bw / aw_sel)
        th = _log16(bh / ah_sel)
        dup_cell = iota < 0
        dup_pair = iota < 0
        for j in range(1, NB):
            samek = (key == key[j]) & (iota < j)
            dup_cell = dup_cell | samek
            dup_pair = dup_pair | (samek & (lbv == lbv[j]))
        ucf = jnp.where(dup_cell, 0.0, 1.0)
        upf = jnp.where(dup_pair, 0.0, 1.0)
        for c in copies:
            c.wait()
        y0v = jnp.zeros((16,), jnp.float32)
        y1v = jnp.zeros((16,), jnp.float32)
        y2v = jnp.zeros((16,), jnp.float32)
        y3v = jnp.zeros((16,), jnp.float32)
        y4v = jnp.zeros((16,), jnp.float32)
        acc0 = acc0 + upf
        for i in range(NB):
            lane = iota == i
            pos = lbv[i] + 5
            ucf_i = ucf[i]
            upf_i = upf[i]
            ai = aidx[i]
            ssq = jnp.zeros((16,), jnp.float32)
            ylsel = jnp.zeros((16,), jnp.float32)
            for k in range(6):
                off = k * 16 if k < 5 else 69
                ch = rows_v[i, ai, pl.ds(off, 16)]
                if k == 0:
                    y0v = jnp.where(lane, ch[0], y0v)
                    y1v = jnp.where(lane, ch[1], y1v)
                    y2v = jnp.where(lane, ch[2], y2v)
                    y3v = jnp.where(lane, ch[3], y3v)
                    y4v = jnp.where(lane, ch[4], y4v)
                    chm = jnp.where(iota >= 5, ch, 0.0)
                    hit = iota == pos
                elif k == 5:
                    chm = jnp.where(iota >= 11, ch, 0.0)
                    hit = (iota >= 11) & (iota + 69 == pos)
                else:
                    chm = ch
                    hit = iota + off == pos
                ssq = ssq + chm * chm
                ylsel = ylsel + jnp.where(hit, ch, 0.0)
            acc0 = acc0 + ucf_i * ssq - (2.0 * upf_i) * ylsel
        pcx = (1.0 / (1.0 + jnp.exp(-y0v)) + cellx.astype(jnp.float32)) * GS
        pcy = (1.0 / (1.0 + jnp.exp(-y1v)) + celly.astype(jnp.float32)) * GS
        pw = aw_sel * jnp.exp(y2v)
        ph = ah_sel * jnp.exp(y3v)
        px1 = pcx - pw * 0.5
        px2 = pcx + pw * 0.5
        py1 = pcy - ph * 0.5
        py2 = pcy + ph * 0.5
        areap = pw * ph
        ioumax = jnp.full((16,), -1.0, jnp.float32)
        tiou = jnp.zeros((16,), jnp.float32)
        for j in range(NB):
            areagj = areag[j]
            iw = jnp.maximum(jnp.minimum(px2, bx2[j]) - jnp.maximum(px1, bx1[j]), 0.0)
            ih = jnp.maximum(jnp.minimum(py2, by2[j]) - jnp.maximum(py1, by1[j]), 0.0)
            inter = iw * ih
            iouj = inter / (areap + areagj - inter + 1e-12)
            ioumax = jnp.maximum(ioumax, iouj)
            tiou = jnp.where(iota == j, iouj, tiou)
        d0 = y0v - offx
        d1 = y1v - offy
        d2 = y2v - tw
        d3 = y3v - th
        d4 = y4v - tiou
        obj_terms = d0 * d0 + d1 * d1 + d2 * d2 + d3 * d3 + d4 * d4
        acc0 = acc0 + ucf * obj_terms
        acc1 = acc1 + ucf
        mcf = jnp.where(ioumax < THR, ucf, 0.0)
        acc2 = acc2 + mcf * y4v * y4v
        acc3 = acc3 + mcf
        return (acc0, acc1, acc2, acc3)

    z = jnp.zeros((16,), jnp.float32)
    acc0, acc1, acc2, acc3 = lax.fori_loop(0, BPW, one_batch, (z, z, z, z))
    out_v[0, :] = acc0
    out_v[1, :] = acc1
    out_v[2, :] = acc2
    out_v[3, :] = acc3
    pltpu.sync_copy(out_v, out.at[wid])


def _make_sc_obj():
    return functools.partial(
        pl.kernel,
        out_type=jax.ShapeDtypeStruct((NW, 4, 16), jnp.float32),
        mesh=plsc.VectorSubcoreMesh(core_axis_name="c", subcore_axis_name="s",
                                    num_cores=NC, num_subcores=NS),
        scratch_types=[
            pltpu.VMEM((64,), jnp.float32),
            pltpu.VMEM((16,), jnp.int32),
            pltpu.VMEM((32,), jnp.float32),
            pltpu.VMEM((16, 8, CH), jnp.float32),
            pltpu.VMEM((4, 16), jnp.float32),
            pltpu.SemaphoreType.DMA,
        ],
    )(_sc_body)


def _tc_body(t5_ref, boxes_ref, anc_ref, gx_ref, gy_ref, av_ref,
             msk_ref, scp_ref, out_ref, acc_ref):
    b = pl.program_id(0)

    @pl.when(b == 0)
    def _init():
        acc_ref[0] = 0.0
        acc_ref[1] = 0.0

    av = av_ref[...]
    aw = jnp.zeros((27, 128), jnp.float32)
    ah = jnp.zeros((27, 128), jnp.float32)
    for a in range(A):
        s = av == float(a)
        aw = jnp.where(s, anc_ref[a, 0], aw)
        ah = jnp.where(s, anc_ref[a, 1], ah)
    for lb in range(4):
        x0 = t5_ref[lb, 0]
        x1 = t5_ref[lb, 1]
        x2 = t5_ref[lb, 2]
        x3 = t5_ref[lb, 3]
        pi = t5_ref[lb, 4]
        pcx = (jax.nn.sigmoid(x0) + gx_ref[...]) * GS
        pcy = (jax.nn.sigmoid(x1) + gy_ref[...]) * GS
        pw = aw * jnp.exp(x2)
        ph = ah * jnp.exp(x3)
        px1 = pcx - pw * 0.5
        px2 = pcx + pw * 0.5
        py1 = pcy - ph * 0.5
        py2 = pcy + ph * 0.5
        areap = pw * ph
        noobj = msk_ref[...] > 0.5
        rhs0 = THR * areap
        for j in range(NB):
            bx1j = boxes_ref[lb, 0, j]
            by1j = boxes_ref[lb, 1, j]
            bx2j = boxes_ref[lb, 2, j]
            by2j = boxes_ref[lb, 3, j]
            areagj = (bx2j - bx1j) * (by2j - by1j)
            iw = jnp.maximum(jnp.minimum(px2, bx2j) - jnp.maximum(px1, bx1j), 0.0)
            ih = jnp.maximum(jnp.minimum(py2, by2j) - jnp.maximum(py1, by1j), 0.0)
            inter = iw * ih
            noobj = noobj & ((1.0 + THR) * inter < rhs0 + THR * areagj)
        acc_ref[0] += jnp.sum(jnp.where(noobj, pi * pi, 0.0))
        acc_ref[1] += jnp.sum(jnp.where(noobj, 1.0, 0.0))

    @pl.when(b == B // 4 - 1)
    def _fin():
        p = scp_ref[...]
        sobj = jnp.sum(p[:, 0, :])
        n1 = jnp.sum(p[:, 1, :])
        cs = jnp.sum(p[:, 2, :])
        cn = jnp.sum(p[:, 3, :])
        out_ref[0, 0] = sobj / n1 + (acc_ref[0] - cs) / (acc_ref[1] - cn)


def kernel(yolo_output, boxes, labels, anchor):
    t5 = jnp.zeros((B, 5, PADC), jnp.float32).at[:, :, :CELLS].set(
        jnp.moveaxis(yolo_output[..., :5], -1, 1).reshape(B, 5, CELLS)
    ).reshape(B, 5, 27, 128)
    boxes_t = jnp.swapaxes(boxes, 1, 2)
    anc_b = jnp.pad(anchor.T, ((0, 0), (0, 16 - A))).reshape(32)
    k = jnp.arange(PADC, dtype=jnp.int32)
    gxc = ((k // A) % GW).astype(jnp.float32).reshape(27, 128)
    gyc = (k // (GW * A)).astype(jnp.float32).reshape(27, 128)
    avc = (k % A).astype(jnp.float32).reshape(27, 128)
    mskc = (k < CELLS).astype(jnp.float32).reshape(27, 128)

    scp = _make_sc_obj()(yolo_output, boxes_t.reshape(B, 64), labels, anc_b)

    loss = pl.pallas_call(
        _tc_body,
        grid=(B // 4,),
        in_specs=[
            pl.BlockSpec((4, 5, 27, 128), lambda b: (b, 0, 0, 0)),
            pl.BlockSpec((4, 4, NB), lambda b: (b, 0, 0),
                         memory_space=pltpu.SMEM),
            pl.BlockSpec((A, 2), lambda b: (0, 0), memory_space=pltpu.SMEM),
            pl.BlockSpec((27, 128), lambda b: (0, 0)),
            pl.BlockSpec((27, 128), lambda b: (0, 0)),
            pl.BlockSpec((27, 128), lambda b: (0, 0)),
            pl.BlockSpec((27, 128), lambda b: (0, 0)),
            pl.BlockSpec((NW, 4, 16), lambda b: (0, 0, 0)),
        ],
        out_specs=pl.BlockSpec((1, 1), lambda b: (0, 0),
                               memory_space=pltpu.SMEM),
        out_shape=jax.ShapeDtypeStruct((1, 1), jnp.float32),
        scratch_shapes=[pltpu.SMEM((2,), jnp.float32)],
    )(t5, boxes_t, anchor, gxc, gyc, avc, mskc, scp)
    return loss[0, 0]


# speedup vs baseline: 1.2746x; 1.0165x over previous
import functools

import jax
import jax.numpy as jnp
from jax import lax
from jax.experimental import pallas as pl
from jax.experimental.pallas import tpu as pltpu
from jax.experimental.pallas import tpu_sc as plsc

B = 64
GH = GW = 26
A = 5
NB = 16
NCLS = 80
CH = 5 + NCLS
CELLS = GH * GW * A
PADC = 3456
GS = 16.0
SR, SL = 20, 169
THR = 0.4
NC, NS = 2, 16
NW = NC * NS
BPW = B // NW
LN2 = 0.6931471805599453


def _log16(x):
    bits = lax.bitcast_convert_type(x, jnp.int32)
    e = ((bits >> 23) & 0xFF) - 127
    m = lax.bitcast_convert_type((bits & 0x7FFFFF) | (127 << 23), jnp.float32)
    big = m > 1.4142135381698608
    m = jnp.where(big, m * 0.5, m)
    e = (e + jnp.where(big, 1, 0)).astype(jnp.float32)
    z = (m - 1.0) / (m + 1.0)
    z2 = z * z
    p = 1.0 + z2 * (0.333333343 + z2 * (0.2 + z2 * (0.142857149 + z2 * 0.111111112)))
    return e * LN2 + 2.0 * z * p


def _sc_body(y5, boxes_t, labels2, anc_b, out,
             boxes_v, labels_v, anc_v, rows_v, out_v, sem):
    wid = lax.axis_index("s") * NC + lax.axis_index("c")
    iota = lax.iota(jnp.int32, 16)
    pltpu.sync_copy(anc_b, anc_v)
    awv = anc_v[pl.ds(0, 16)]
    ahv = anc_v[pl.ds(16, 16)]

    def one_batch(lb, accs):
        acc0, acc1, acc2, acc3 = accs
        b = wid * BPW + lb
        pltpu.sync_copy(boxes_t.at[b], boxes_v)
        pltpu.sync_copy(labels2.at[b], labels_v)
        bx1 = boxes_v[pl.ds(0, 16)]
        by1 = boxes_v[pl.ds(16, 16)]
        bx2 = boxes_v[pl.ds(32, 16)]
        by2 = boxes_v[pl.ds(48, 16)]
        lbv = labels_v[:]
        bw = bx2 - bx1
        bh = by2 - by1
        areag = bw * bh
        cxg = (bx1 + bx2) * (0.5 / GS)
        cyg = (by1 + by2) * (0.5 / GS)
        cellx = cxg.astype(jnp.int32)
        celly = cyg.astype(jnp.int32)
        offx = cxg - cellx.astype(jnp.float32)
        offy = cyg - celly.astype(jnp.float32)
        best = jnp.full((16,), -1.0, jnp.float32)
        aw_sel = jnp.zeros((16,), jnp.float32)
        ah_sel = jnp.zeros((16,), jnp.float32)
        aidx = jnp.zeros((16,), jnp.int32)
        for a in range(A):
            awa = awv[a]
            aha = ahv[a]
            inter = jnp.minimum(awa, bw) * jnp.minimum(aha, bh)
            ioua = inter / (awa * aha + areag - inter + 1e-12)
            take = ioua > best
            best = jnp.where(take, ioua, best)
            aidx = jnp.where(take, a, aidx)
            aw_sel = jnp.where(take, awa, aw_sel)
            ah_sel = jnp.where(take, aha, ah_sel)
        key = (celly * GW + cellx) * A + aidx
        copies = []
        for i in range(NB):
            copies.append(pltpu.async_copy(
                y5.at[b, celly[i], cellx[i]], rows_v.at[i, pl.ds(0, A)], sem))
        tw = _log16(bw / aw_sel)
        th = _log16(bh / ah_sel)
        dup_cell = iota < 0
        dup_pair = iota < 0
        for j in range(1, NB):
            samek = (key == key[j]) & (iota < j)
            dup_cell = dup_cell | samek
            dup_pair = dup_pair | (samek & (lbv == lbv[j]))
        ucf = jnp.where(dup_cell, 0.0, 1.0)
        upf = jnp.where(dup_pair, 0.0, 1.0)
        for c in copies:
            c.wait()
        y0v = jnp.zeros((16,), jnp.float32)
        y1v = jnp.zeros((16,), jnp.float32)
        y2v = jnp.zeros((16,), jnp.float32)
        y3v = jnp.zeros((16,), jnp.float32)
        y4v = jnp.zeros((16,), jnp.float32)
        acc0 = acc0 + upf
        for i in range(NB):
            lane = iota == i
            pos = lbv[i] + 5
            ucf_i = ucf[i]
            upf_i = upf[i]
            ai = aidx[i]
            ssq = jnp.zeros((16,), jnp.float32)
            ylsel = jnp.zeros((16,), jnp.float32)
            for k in range(6):
                off = k * 16 if k < 5 else 69
                ch = rows_v[i, ai, pl.ds(off, 16)]
                if k == 0:
                    y0v = jnp.where(lane, ch[0], y0v)
                    y1v = jnp.where(lane, ch[1], y1v)
                    y2v = jnp.where(lane, ch[2], y2v)
                    y3v = jnp.where(lane, ch[3], y3v)
                    y4v = jnp.where(lane, ch[4], y4v)
                    chm = jnp.where(iota >= 5, ch, 0.0)
                    hit = iota == pos
                elif k == 5:
                    chm = jnp.where(iota >= 11, ch, 0.0)
                    hit = (iota >= 11) & (iota + 69 == pos)
                else:
                    chm = ch
                    hit = iota + off == pos
                ssq = ssq + chm * chm
                ylsel = ylsel + jnp.where(hit, ch, 0.0)
            acc0 = acc0 + ucf_i * ssq - (2.0 * upf_i) * ylsel
        pcx = (1.0 / (1.0 + jnp.exp(-y0v)) + cellx.astype(jnp.float32)) * GS
        pcy = (1.0 / (1.0 + jnp.exp(-y1v)) + celly.astype(jnp.float32)) * GS
        pw = aw_sel * jnp.exp(y2v)
        ph = ah_sel * jnp.exp(y3v)
        px1 = pcx - pw * 0.5
        px2 = pcx + pw * 0.5
        py1 = pcy - ph * 0.5
        py2 = pcy + ph * 0.5
        areap = pw * ph
        ioumax = jnp.full((16,), -1.0, jnp.float32)
        tiou = jnp.zeros((16,), jnp.float32)
        for j in range(NB):
            areagj = areag[j]
            iw = jnp.maximum(jnp.minimum(px2, bx2[j]) - jnp.maximum(px1, bx1[j]), 0.0)
            ih = jnp.maximum(jnp.minimum(py2, by2[j]) - jnp.maximum(py1, by1[j]), 0.0)
            inter = iw * ih
            iouj = inter / (areap + areagj - inter + 1e-12)
            ioumax = jnp.maximum(ioumax, iouj)
            tiou = jnp.where(iota == j, iouj, tiou)
        d0 = y0v - offx
        d1 = y1v - offy
        d2 = y2v - tw
        d3 = y3v - th
        d4 = y4v - tiou
        obj_terms = d0 * d0 + d1 * d1 + d2 * d2 + d3 * d3 + d4 * d4
        acc0 = acc0 + ucf * obj_terms
        acc1 = acc1 + ucf
        mcf = jnp.where(ioumax < THR, ucf, 0.0)
        acc2 = acc2 + mcf * y4v * y4v
        acc3 = acc3 + mcf
        return (acc0, acc1, acc2, acc3)

    z = jnp.zeros((16,), jnp.float32)
    acc0, acc1, acc2, acc3 = lax.fori_loop(0, BPW, one_batch, (z, z, z, z))
    out_v[0, :] = acc0
    out_v[1, :] = acc1
    out_v[2, :] = acc2
    out_v[3, :] = acc3
    pltpu.sync_copy(out_v, out.at[wid])


def _make_sc_obj():
    return functools.partial(
        pl.kernel,
        out_type=jax.ShapeDtypeStruct((NW, 4, 16), jnp.float32),
        mesh=plsc.VectorSubcoreMesh(core_axis_name="c", subcore_axis_name="s",
                                    num_cores=NC, num_subcores=NS),
        scratch_types=[
            pltpu.VMEM((64,), jnp.float32),
            pltpu.VMEM((16,), jnp.int32),
            pltpu.VMEM((32,), jnp.float32),
            pltpu.VMEM((16, 8, CH), jnp.float32),
            pltpu.VMEM((4, 16), jnp.float32),
            pltpu.SemaphoreType.DMA,
        ],
    )(_sc_body)


def _tc_body(t5_ref, boxes_ref, anc_ref, gx_ref, gy_ref, av_ref,
             msk_ref, scp_ref, out_ref, acc_ref):
    b = pl.program_id(0)

    @pl.when(b == 0)
    def _init():
        acc_ref[0] = 0.0
        acc_ref[1] = 0.0

    av = av_ref[...]
    aw = jnp.zeros((27, 128), jnp.float32)
    ah = jnp.zeros((27, 128), jnp.float32)
    for a in range(A):
        s = av == float(a)
        aw = jnp.where(s, anc_ref[a, 0], aw)
        ah = jnp.where(s, anc_ref[a, 1], ah)
    for lb in range(8):
        x0 = t5_ref[lb, 0]
        x1 = t5_ref[lb, 1]
        x2 = t5_ref[lb, 2]
        x3 = t5_ref[lb, 3]
        pi = t5_ref[lb, 4]
        pcx = (jax.nn.sigmoid(x0) + gx_ref[...]) * GS
        pcy = (jax.nn.sigmoid(x1) + gy_ref[...]) * GS
        pw = aw * jnp.exp(x2)
        ph = ah * jnp.exp(x3)
        px1 = pcx - pw * 0.5
        px2 = pcx + pw * 0.5
        py1 = pcy - ph * 0.5
        py2 = pcy + ph * 0.5
        areap = pw * ph
        noobj = msk_ref[...] > 0.5
        rhs0 = THR * areap
        for j in range(NB):
            bx1j = boxes_ref[lb, 0, j]
            by1j = boxes_ref[lb, 1, j]
            bx2j = boxes_ref[lb, 2, j]
            by2j = boxes_ref[lb, 3, j]
            areagj = (bx2j - bx1j) * (by2j - by1j)
            iw = jnp.maximum(jnp.minimum(px2, bx2j) - jnp.maximum(px1, bx1j), 0.0)
            ih = jnp.maximum(jnp.minimum(py2, by2j) - jnp.maximum(py1, by1j), 0.0)
            inter = iw * ih
            noobj = noobj & ((1.0 + THR) * inter < rhs0 + THR * areagj)
        acc_ref[0] += jnp.sum(jnp.where(noobj, pi * pi, 0.0))
        acc_ref[1] += jnp.sum(jnp.where(noobj, 1.0, 0.0))

    @pl.when(b == B // 8 - 1)
    def _fin():
        p = scp_ref[...]
        sobj = jnp.sum(p[:, 0, :])
        n1 = jnp.sum(p[:, 1, :])
        cs = jnp.sum(p[:, 2, :])
        cn = jnp.sum(p[:, 3, :])
        out_ref[0, 0] = sobj / n1 + (acc_ref[0] - cs) / (acc_ref[1] - cn)


def kernel(yolo_output, boxes, labels, anchor):
    t5 = jnp.zeros((B, 5, PADC), jnp.float32).at[:, :, :CELLS].set(
        jnp.moveaxis(yolo_output[..., :5], -1, 1).reshape(B, 5, CELLS)
    ).reshape(B, 5, 27, 128)
    boxes_t = jnp.swapaxes(boxes, 1, 2)
    anc_b = jnp.pad(anchor.T, ((0, 0), (0, 16 - A))).reshape(32)
    k = jnp.arange(PADC, dtype=jnp.int32)
    gxc = ((k // A) % GW).astype(jnp.float32).reshape(27, 128)
    gyc = (k // (GW * A)).astype(jnp.float32).reshape(27, 128)
    avc = (k % A).astype(jnp.float32).reshape(27, 128)
    mskc = (k < CELLS).astype(jnp.float32).reshape(27, 128)

    scp = _make_sc_obj()(yolo_output, boxes_t.reshape(B, 64), labels, anc_b)

    loss = pl.pallas_call(
        _tc_body,
        grid=(B // 8,),
        in_specs=[
            pl.BlockSpec((8, 5, 27, 128), lambda b: (b, 0, 0, 0)),
            pl.BlockSpec((8, 4, NB), lambda b: (b, 0, 0),
                         memory_space=pltpu.SMEM),
            pl.BlockSpec((A, 2), lambda b: (0, 0), memory_space=pltpu.SMEM),
            pl.BlockSpec((27, 128), lambda b: (0, 0)),
            pl.BlockSpec((27, 128), lambda b: (0, 0)),
            pl.BlockSpec((27, 128), lambda b: (0, 0)),
            pl.BlockSpec((27, 128), lambda b: (0, 0)),
            pl.BlockSpec((NW, 4, 16), lambda b: (0, 0, 0)),
        ],
        out_specs=pl.BlockSpec((1, 1), lambda b: (0, 0),
                               memory_space=pltpu.SMEM),
        out_shape=jax.ShapeDtypeStruct((1, 1), jnp.float32),
        scratch_shapes=[pltpu.SMEM((2,), jnp.float32)],
    )(t5, boxes_t, anchor, gxc, gyc, avc, mskc, scp)
    return loss[0, 0]
